# baseline (speedup 1.0000x reference)
import jax
import jax.numpy as jnp
from jax import lax
from jax.experimental import pallas as pl
from jax.experimental.pallas import tpu as pltpu
from jax.experimental.pallas import tpu_sc as plsc

_NC, _NS = 2, 16
_NW = _NC * _NS
_T, _D, _B = 8192, 1024, 4
_TS = 2560
_RB = 8
_TPW = _TS // _NW
_NB = _TPW // _RB
_TB = 512


def _sc_body(x_hbm, e_hbm, o_hbm,
             xa0, xa1, xa2, xa3, ea,
             xb0, xb1, xb2, xb3, eb,
             isa, esa, osa, isb, esb, osb):
    xbufs = ((xa0, xa1, xa2, xa3), (xb0, xb1, xb2, xb3))
    ebufs = (ea, eb)
    isems = (isa, isb)
    esems = (esa, esb)
    osems = (osa, osb)
    w = lax.axis_index("s") * _NC + lax.axis_index("c")
    t0 = w * _TPW

    def x_rows(g, j):
        return pl.ds(j * _T + t0 + g * _RB, _RB)

    def o_rows(g, j):
        return pl.ds(j * _TS + t0 + g * _RB, _RB)

    def e_rows(g):
        return pl.ds(t0 + g * _RB, _RB)

    def start_in(g, p):
        for j in range(_B):
            pltpu.async_copy(x_hbm.at[x_rows(g, j)], xbufs[p][j], isems[p])
        pltpu.async_copy(e_hbm.at[e_rows(g)], ebufs[p], esems[p])

    def wait_in(g, p):
        for j in range(_B):
            pltpu.make_async_copy(
                x_hbm.at[x_rows(g, j)], xbufs[p][j], isems[p]).wait()
        pltpu.make_async_copy(e_hbm.at[e_rows(g)], ebufs[p], esems[p]).wait()

    def wait_out(g, p):
        for j in range(_B):
            pltpu.make_async_copy(
                xbufs[p][j], o_hbm.at[o_rows(g, j)], osems[p]).wait()

    def add_block(p, j):
        buf = xbufs[p][j]
        ebf = ebufs[p]
        for r in range(_RB):
            @plsc.parallel_loop(0, _D, step=16, unroll=8)
            def _add(i):
                s = pl.ds(i, 16)
                buf[r, s] = buf[r, s] + ebf[r, s]

    start_in(0, 0)

    def body(gg, carry):
        for par in range(2):
            g = 2 * gg + par
            q = 1 - par
            wait_in(g, par)
            add_block(par, 0)
            pltpu.async_copy(xbufs[par][0], o_hbm.at[o_rows(g, 0)], osems[par])
            @pl.when(g > 0)
            def _():
                wait_out(g - 1, q)
            @pl.when(g + 1 < _NB)
            def _():
                start_in(g + 1, q)
            for j in range(1, _B):
                add_block(par, j)
                pltpu.async_copy(
                    xbufs[par][j], o_hbm.at[o_rows(g, j)], osems[par])
        return carry

    lax.fori_loop(0, _NB // 2, body, 0)
    wait_out(_NB - 1, 1)


def _tc_body(x_ref, e_ref, o_ref):
    o_ref[...] = x_ref[...] + e_ref[...][None, :, :]


def kernel(x, embedding):
    B, T, D = x.shape
    x2 = x.reshape(B * T, D)

    sc_run = pl.kernel(
        _sc_body,
        out_type=jax.ShapeDtypeStruct((B * _TS, D), x.dtype),
        mesh=plsc.VectorSubcoreMesh(
            core_axis_name="c", subcore_axis_name="s",
            num_cores=_NC, num_subcores=_NS,
        ),
        scratch_types=(
            [pltpu.VMEM((_RB, _D), jnp.float32)] * 5
            + [pltpu.VMEM((_RB, _D), jnp.float32)] * 5
            + [pltpu.SemaphoreType.DMA] * 6
        ),
        compiler_params=pltpu.CompilerParams(use_tc_tiling_on_sc=True),
    )
    sc_part = sc_run(x2, embedding)

    ntc = (T - _TS) // _TB
    off = _TS // _TB
    tc_full = pl.pallas_call(
        _tc_body,
        grid=(ntc,),
        in_specs=[
            pl.BlockSpec((B, _TB, D), lambda i: (0, i + off, 0)),
            pl.BlockSpec((_TB, D), lambda i: (i + off, 0)),
        ],
        out_specs=pl.BlockSpec((B, _TB, D), lambda i: (0, i + off, 0)),
        out_shape=jax.ShapeDtypeStruct((B, T, D), x.dtype),
        compiler_params=pltpu.CompilerParams(
            dimension_semantics=("arbitrary",),
        ),
    )(x, embedding)

    return lax.dynamic_update_slice(
        tc_full, sc_part.reshape(B, _TS, D), (0, 0, 0))


# speedup vs baseline: 1.1409x; 1.1409x over previous
import jax
import jax.numpy as jnp
from jax import lax
from jax.experimental import pallas as pl
from jax.experimental.pallas import tpu as pltpu
from jax.experimental.pallas import tpu_sc as plsc

_NC, _NS = 2, 16
_NW = _NC * _NS
_T, _D, _B = 8192, 1024, 4
_TS = 1024
_RB = 8
_TPW = _TS // _NW
_NB = _TPW // _RB
_TB = 512


def _sc_body(x_hbm, e_hbm, o_hbm,
             xa0, xa1, xa2, xa3, ea,
             xb0, xb1, xb2, xb3, eb,
             isa, esa, osa, isb, esb, osb):
    xbufs = ((xa0, xa1, xa2, xa3), (xb0, xb1, xb2, xb3))
    ebufs = (ea, eb)
    isems = (isa, isb)
    esems = (esa, esb)
    osems = (osa, osb)
    w = lax.axis_index("s") * _NC + lax.axis_index("c")
    t0 = w * _TPW

    def x_rows(g, j):
        return pl.ds(j * _T + t0 + g * _RB, _RB)

    def o_rows(g, j):
        return pl.ds(j * _TS + t0 + g * _RB, _RB)

    def e_rows(g):
        return pl.ds(t0 + g * _RB, _RB)

    def start_in(g, p):
        for j in range(_B):
            pltpu.async_copy(x_hbm.at[x_rows(g, j)], xbufs[p][j], isems[p])
        pltpu.async_copy(e_hbm.at[e_rows(g)], ebufs[p], esems[p])

    def wait_in(g, p):
        for j in range(_B):
            pltpu.make_async_copy(
                x_hbm.at[x_rows(g, j)], xbufs[p][j], isems[p]).wait()
        pltpu.make_async_copy(e_hbm.at[e_rows(g)], ebufs[p], esems[p]).wait()

    def wait_out(g, p):
        for j in range(_B):
            pltpu.make_async_copy(
                xbufs[p][j], o_hbm.at[o_rows(g, j)], osems[p]).wait()

    def add_block(p, j):
        buf = xbufs[p][j]
        ebf = ebufs[p]
        for r in range(_RB):
            @plsc.parallel_loop(0, _D, step=16, unroll=8)
            def _add(i):
                s = pl.ds(i, 16)
                buf[r, s] = buf[r, s] + ebf[r, s]

    start_in(0, 0)

    def body(gg, carry):
        for par in range(2):
            g = 2 * gg + par
            q = 1 - par
            wait_in(g, par)
            add_block(par, 0)
            pltpu.async_copy(xbufs[par][0], o_hbm.at[o_rows(g, 0)], osems[par])
            @pl.when(g > 0)
            def _():
                wait_out(g - 1, q)
            @pl.when(g + 1 < _NB)
            def _():
                start_in(g + 1, q)
            for j in range(1, _B):
                add_block(par, j)
                pltpu.async_copy(
                    xbufs[par][j], o_hbm.at[o_rows(g, j)], osems[par])
        return carry

    lax.fori_loop(0, _NB // 2, body, 0)
    wait_out(_NB - 1, 1)


def _tc_body(x_ref, e_ref, o_ref):
    o_ref[...] = x_ref[...] + e_ref[...][None, :, :]


def kernel(x, embedding):
    B, T, D = x.shape
    x2 = x.reshape(B * T, D)

    sc_run = pl.kernel(
        _sc_body,
        out_type=jax.ShapeDtypeStruct((B * _TS, D), x.dtype),
        mesh=plsc.VectorSubcoreMesh(
            core_axis_name="c", subcore_axis_name="s",
            num_cores=_NC, num_subcores=_NS,
        ),
        scratch_types=(
            [pltpu.VMEM((_RB, _D), jnp.float32)] * 5
            + [pltpu.VMEM((_RB, _D), jnp.float32)] * 5
            + [pltpu.SemaphoreType.DMA] * 6
        ),
        compiler_params=pltpu.CompilerParams(use_tc_tiling_on_sc=True),
    )
    sc_part = sc_run(x2, embedding)

    ntc = (T - _TS) // _TB
    off = _TS // _TB
    tc_full = pl.pallas_call(
        _tc_body,
        grid=(ntc,),
        in_specs=[
            pl.BlockSpec((B, _TB, D), lambda i: (0, i + off, 0)),
            pl.BlockSpec((_TB, D), lambda i: (i + off, 0)),
        ],
        out_specs=pl.BlockSpec((B, _TB, D), lambda i: (0, i + off, 0)),
        out_shape=jax.ShapeDtypeStruct((B, T, D), x.dtype),
        compiler_params=pltpu.CompilerParams(
            dimension_semantics=("arbitrary",),
        ),
    )(x, embedding)

    return lax.dynamic_update_slice(
        tc_full, sc_part.reshape(B, _TS, D), (0, 0, 0))


# speedup vs baseline: 1.1866x; 1.0400x over previous
import jax
import jax.numpy as jnp
from jax import lax
from jax.experimental import pallas as pl
from jax.experimental.pallas import tpu as pltpu
from jax.experimental.pallas import tpu_sc as plsc

_NC, _NS = 2, 16
_NW = _NC * _NS
_T, _D, _B = 8192, 1024, 4
_TS = 512
_RB = 8
_TPW = _TS // _NW
_NB = _TPW // _RB
_TB = 512


def _sc_body(x_hbm, e_hbm, o_hbm,
             xa0, xa1, xa2, xa3, ea,
             xb0, xb1, xb2, xb3, eb,
             isa, esa, osa, isb, esb, osb):
    xbufs = ((xa0, xa1, xa2, xa3), (xb0, xb1, xb2, xb3))
    ebufs = (ea, eb)
    isems = (isa, isb)
    esems = (esa, esb)
    osems = (osa, osb)
    w = lax.axis_index("s") * _NC + lax.axis_index("c")
    t0 = w * _TPW

    def x_rows(g, j):
        return pl.ds(j * _T + t0 + g * _RB, _RB)

    def o_rows(g, j):
        return pl.ds(j * _TS + t0 + g * _RB, _RB)

    def e_rows(g):
        return pl.ds(t0 + g * _RB, _RB)

    def start_in(g, p):
        for j in range(_B):
            pltpu.async_copy(x_hbm.at[x_rows(g, j)], xbufs[p][j], isems[p])
        pltpu.async_copy(e_hbm.at[e_rows(g)], ebufs[p], esems[p])

    def wait_in(g, p):
        for j in range(_B):
            pltpu.make_async_copy(
                x_hbm.at[x_rows(g, j)], xbufs[p][j], isems[p]).wait()
        pltpu.make_async_copy(e_hbm.at[e_rows(g)], ebufs[p], esems[p]).wait()

    def wait_out(g, p):
        for j in range(_B):
            pltpu.make_async_copy(
                xbufs[p][j], o_hbm.at[o_rows(g, j)], osems[p]).wait()

    def add_block(p, j):
        buf = xbufs[p][j]
        ebf = ebufs[p]
        for r in range(_RB):
            @plsc.parallel_loop(0, _D, step=16, unroll=8)
            def _add(i):
                s = pl.ds(i, 16)
                buf[r, s] = buf[r, s] + ebf[r, s]

    start_in(0, 0)

    def body(gg, carry):
        for par in range(2):
            g = 2 * gg + par
            q = 1 - par
            wait_in(g, par)
            add_block(par, 0)
            pltpu.async_copy(xbufs[par][0], o_hbm.at[o_rows(g, 0)], osems[par])
            @pl.when(g > 0)
            def _():
                wait_out(g - 1, q)
            @pl.when(g + 1 < _NB)
            def _():
                start_in(g + 1, q)
            for j in range(1, _B):
                add_block(par, j)
                pltpu.async_copy(
                    xbufs[par][j], o_hbm.at[o_rows(g, j)], osems[par])
        return carry

    lax.fori_loop(0, _NB // 2, body, 0)
    wait_out(_NB - 1, 1)


def _tc_body(x_ref, e_ref, o_ref):
    o_ref[...] = x_ref[...] + e_ref[...][None, :, :]


def kernel(x, embedding):
    B, T, D = x.shape
    x2 = x.reshape(B * T, D)

    sc_run = pl.kernel(
        _sc_body,
        out_type=jax.ShapeDtypeStruct((B * _TS, D), x.dtype),
        mesh=plsc.VectorSubcoreMesh(
            core_axis_name="c", subcore_axis_name="s",
            num_cores=_NC, num_subcores=_NS,
        ),
        scratch_types=(
            [pltpu.VMEM((_RB, _D), jnp.float32)] * 5
            + [pltpu.VMEM((_RB, _D), jnp.float32)] * 5
            + [pltpu.SemaphoreType.DMA] * 6
        ),
        compiler_params=pltpu.CompilerParams(use_tc_tiling_on_sc=True),
    )
    sc_part = sc_run(x2, embedding)

    ntc = (T - _TS) // _TB
    off = _TS // _TB
    tc_full = pl.pallas_call(
        _tc_body,
        grid=(ntc,),
        in_specs=[
            pl.BlockSpec((B, _TB, D), lambda i: (0, i + off, 0)),
            pl.BlockSpec((_TB, D), lambda i: (i + off, 0)),
        ],
        out_specs=pl.BlockSpec((B, _TB, D), lambda i: (0, i + off, 0)),
        out_shape=jax.ShapeDtypeStruct((B, T, D), x.dtype),
        compiler_params=pltpu.CompilerParams(
            dimension_semantics=("arbitrary",),
        ),
    )(x, embedding)

    return lax.dynamic_update_slice(
        tc_full, sc_part.reshape(B, _TS, D), (0, 0, 0))
